# baseline (speedup 1.0000x reference)
import jax, jax.numpy as jnp
from jax.experimental import pallas as pl

NH, E = 16, 8
HP = jax.lax.Precision.HIGHEST


def _layer_norm(x, g, b):
    m = jnp.mean(x, axis=-1, keepdims=True)
    v = jnp.var(x, axis=-1, keepdims=True)
    return (x - m) / jnp.sqrt(v + 1e-5) * g + b


def kernel(x, ln1_g, ln1_b, in_proj_w, in_proj_b, out_proj_w, out_proj_b,
           ln2_g, ln2_b, router_w, w1, b1, w2, b2):
    Bq, Sq, Hq = x.shape
    dh = Hq // NH
    h = _layer_norm(x, ln1_g, ln1_b)
    qkv = jnp.dot(h, in_proj_w.T, precision=HP) + in_proj_b
    q, k, v = jnp.split(qkv, 3, axis=-1)
    def heads(t):
        return t.reshape(Bq, Sq, NH, dh).transpose(0, 2, 1, 3)
    q, k, v = heads(q), heads(k), heads(v)
    scores = jnp.einsum('bhsd,bhtd->bhst', q, k, precision=HP) / jnp.sqrt(jnp.float32(dh))
    causal = jnp.tril(jnp.ones((Sq, Sq), dtype=bool))
    scores = jnp.where(causal[None, None, :, :], scores, jnp.float32(-1e9))
    probs = jax.nn.softmax(scores, axis=-1)
    attn = jnp.einsum('bhst,bhtd->bhsd', probs, v, precision=HP).transpose(0, 2, 1, 3).reshape(Bq, Sq, Hq)
    attn_out = jnp.dot(attn, out_proj_w.T, precision=HP) + out_proj_b
    x = x + attn_out
    h2 = _layer_norm(x, ln2_g, ln2_b)
    T = Bq * Sq
    hf = h2.reshape(T, Hq)
    logits = jnp.dot(hf, router_w.T, precision=HP)
    gprobs = jax.nn.softmax(logits, axis=-1)
    idx = jnp.argmax(gprobs, axis=-1)
    gate = jnp.take_along_axis(gprobs, idx[:, None], axis=1)[:, 0]
    C = T // E
    oh = jax.nn.one_hot(idx, E, dtype=jnp.float32)
    pos = jnp.sum((jnp.cumsum(oh, axis=0) - 1.0) * oh, axis=1).astype(jnp.int32)
    keep = (pos < C).astype(jnp.float32)
    pos_oh = jax.nn.one_hot(pos, C, dtype=jnp.float32)
    dispatch = oh[:, :, None] * pos_oh[:, None, :] * keep[:, None, None]
    combine = dispatch * gate[:, None, None]
    disp = jnp.einsum('tec,th->ech', dispatch, hf, precision=HP)
    mid = jax.nn.gelu(jnp.einsum('ech,efh->ecf', disp, w1, precision=HP) + b1[:, None, :], approximate=False)
    eout = jnp.einsum('ecf,ehf->ech', mid, w2, precision=HP) + b2[:, None, :]
    moe_out = jnp.einsum('tec,ech->th', combine, eout, precision=HP).reshape(Bq, Sq, Hq)
    return x + moe_out


# speedup vs baseline: 2.7103x; 2.7103x over previous
import jax
import jax.numpy as jnp
from jax.experimental import pallas as pl
from jax.experimental.pallas import tpu as pltpu

NH_ = 16
E_ = 8


def _bdot(a, b):
    return jax.lax.dot(a.astype(jnp.bfloat16), b.astype(jnp.bfloat16),
                       preferred_element_type=jnp.float32)


def _ln(x, g, b):
    m = jnp.mean(x, axis=-1, keepdims=True)
    xm = x - m
    v = jnp.mean(xm * xm, axis=-1, keepdims=True)
    return xm / jnp.sqrt(v + 1e-5) * g + b


def _qkv_kernel(x_ref, g_ref, b_ref, wT_ref, pb_ref, o_ref):
    h = _ln(x_ref[...], g_ref[...], b_ref[...])
    acc = jax.lax.dot(h.astype(jnp.bfloat16), wT_ref[...],
                      preferred_element_type=jnp.float32)
    o_ref[...] = (acc + pb_ref[...]).astype(jnp.bfloat16)


def _attn_kernel(q_ref, kT_ref, v_ref, o_ref):
    j = pl.program_id(1)
    s = jax.lax.dot(q_ref[0], kT_ref[0],
                    preferred_element_type=jnp.float32) * 0.125
    row = j * 512 + jax.lax.broadcasted_iota(jnp.int32, s.shape, 0)
    col = jax.lax.broadcasted_iota(jnp.int32, s.shape, 1)
    s = jnp.where(col <= row, s, -1e9)
    m = jnp.max(s, axis=-1, keepdims=True)
    p = jnp.exp(s - m)
    probs = p / jnp.sum(p, axis=-1, keepdims=True)
    o_ref[0] = jax.lax.dot(probs.astype(jnp.bfloat16), v_ref[0],
                           preferred_element_type=jnp.float32).astype(jnp.bfloat16)


def _proj_route_kernel(x_ref, a_ref, woT_ref, ob_ref, g2_ref, b2_ref, rT_ref,
                       x2_ref, h2b_ref, slot_ref, gate_ref, cnt_ref):
    i = pl.program_id(0)

    @pl.when(i == 0)
    def _():
        cnt_ref[...] = jnp.zeros_like(cnt_ref)

    x2 = x_ref[...] + (jax.lax.dot(a_ref[...], woT_ref[...],
                                   preferred_element_type=jnp.float32)
                       + ob_ref[...])
    x2_ref[...] = x2
    h2 = _ln(x2, g2_ref[...], b2_ref[...])
    h2b = h2.astype(jnp.bfloat16)
    h2b_ref[...] = h2b
    logits = jax.lax.dot(h2b, rT_ref[...],
                         preferred_element_type=jnp.float32)
    m = jnp.max(logits, axis=-1, keepdims=True)
    p = jnp.exp(logits - m)
    gp = p / jnp.sum(p, axis=-1, keepdims=True)
    gmax = jnp.max(gp, axis=-1, keepdims=True)
    e_io = jax.lax.broadcasted_iota(jnp.int32, gp.shape, 1)
    idx = jnp.min(jnp.where(gp == gmax, e_io, E_), axis=-1, keepdims=True)
    oh = (e_io == idx).astype(jnp.float32)
    r_io = jax.lax.broadcasted_iota(jnp.int32, (512, 512), 0)
    c_io = jax.lax.broadcasted_iota(jnp.int32, (512, 512), 1)
    tril = (c_io <= r_io).astype(jnp.bfloat16)
    cum = jax.lax.dot(tril, oh.astype(jnp.bfloat16),
                      preferred_element_type=jnp.float32)
    carry = cnt_ref[...]
    pos = jnp.sum((cum - 1.0 + carry) * oh, axis=-1, keepdims=True)
    pos = pos.astype(jnp.int32)
    cnt_ref[...] = carry + jnp.sum(oh, axis=0, keepdims=True)
    keep = pos < 512
    slot_ref[...] = jnp.where(keep, idx * 512 + pos, -1)
    gate_ref[...] = gmax


def _dispatch_kernel(slot_ref, h2b_ref, disp_ref):
    e = pl.program_id(0)
    slot = slot_ref[...]
    s_io = jax.lax.broadcasted_iota(jnp.int32, (slot.shape[0], 512), 1) + e * 512
    mask = (slot == s_io).astype(jnp.bfloat16)
    acc = jax.lax.dot_general(mask, h2b_ref[...],
                              (((0,), (0,)), ((), ())),
                              preferred_element_type=jnp.float32)
    disp_ref[...] = acc.astype(jnp.bfloat16)


def _ffn_kernel(disp_ref, w1T_ref, b1_ref, w2T_ref, b2_ref, eout_ref):
    mid = jax.lax.dot(disp_ref[...], w1T_ref[0],
                      preferred_element_type=jnp.float32) + b1_ref[0]
    mid = 0.5 * mid * (1.0 + jax.lax.erf(mid * (2.0 ** -0.5)))
    out = jax.lax.dot(mid.astype(jnp.bfloat16), w2T_ref[0],
                      preferred_element_type=jnp.float32) + b2_ref[0]
    eout_ref[...] = out.astype(jnp.bfloat16)


def _combine_kernel(x2_ref, slot_ref, gate_ref, eout_ref, o_ref):
    slot = slot_ref[...]
    s_io = jax.lax.broadcasted_iota(jnp.int32, (512, eout_ref.shape[0]), 1)
    mask = ((slot == s_io).astype(jnp.float32) * gate_ref[...]).astype(jnp.bfloat16)
    pick = jax.lax.dot(mask, eout_ref[...], preferred_element_type=jnp.float32)
    o_ref[...] = x2_ref[...] + pick


def kernel(x, ln1_g, ln1_b, in_proj_w, in_proj_b, out_proj_w, out_proj_b,
           ln2_g, ln2_b, router_w, w1, b1, w2, b2):
    B, S, H = x.shape
    T = B * S
    dh = H // NH_
    C = T // E_
    FF = w1.shape[1]
    f32 = jnp.float32
    bf16 = jnp.bfloat16
    xf = x.reshape(T, H)

    qkv = pl.pallas_call(
        _qkv_kernel,
        grid=(T // 512,),
        in_specs=[
            pl.BlockSpec((512, H), lambda i: (i, 0)),
            pl.BlockSpec((1, H), lambda i: (0, 0)),
            pl.BlockSpec((1, H), lambda i: (0, 0)),
            pl.BlockSpec((H, 3 * H), lambda i: (0, 0)),
            pl.BlockSpec((1, 3 * H), lambda i: (0, 0)),
        ],
        out_specs=pl.BlockSpec((512, 3 * H), lambda i: (i, 0)),
        out_shape=jax.ShapeDtypeStruct((T, 3 * H), bf16),
    )(xf, ln1_g.reshape(1, H), ln1_b.reshape(1, H),
      in_proj_w.T.astype(bf16), in_proj_b.reshape(1, 3 * H))

    q, k, v = jnp.split(qkv, 3, axis=-1)

    def heads(t):
        return t.reshape(B, S, NH_, dh).transpose(0, 2, 1, 3).reshape(B * NH_, S, dh)

    qh, kh, vh = heads(q), heads(k), heads(v)
    kT = kh.transpose(0, 2, 1)

    attn = pl.pallas_call(
        _attn_kernel,
        grid=(B * NH_, S // 512),
        in_specs=[
            pl.BlockSpec((1, 512, dh), lambda i, j: (i, j, 0)),
            pl.BlockSpec((1, dh, S), lambda i, j: (i, 0, 0)),
            pl.BlockSpec((1, S, dh), lambda i, j: (i, 0, 0)),
        ],
        out_specs=pl.BlockSpec((1, 512, dh), lambda i, j: (i, j, 0)),
        out_shape=jax.ShapeDtypeStruct((B * NH_, S, dh), bf16),
    )(qh, kT, vh)

    attn2 = attn.reshape(B, NH_, S, dh).transpose(0, 2, 1, 3).reshape(T, H)

    x2, h2b, slot, gate = pl.pallas_call(
        _proj_route_kernel,
        grid=(T // 512,),
        in_specs=[
            pl.BlockSpec((512, H), lambda i: (i, 0)),
            pl.BlockSpec((512, H), lambda i: (i, 0)),
            pl.BlockSpec((H, H), lambda i: (0, 0)),
            pl.BlockSpec((1, H), lambda i: (0, 0)),
            pl.BlockSpec((1, H), lambda i: (0, 0)),
            pl.BlockSpec((1, H), lambda i: (0, 0)),
            pl.BlockSpec((H, E_), lambda i: (0, 0)),
        ],
        out_specs=[
            pl.BlockSpec((512, H), lambda i: (i, 0)),
            pl.BlockSpec((512, H), lambda i: (i, 0)),
            pl.BlockSpec((512, 1), lambda i: (i, 0)),
            pl.BlockSpec((512, 1), lambda i: (i, 0)),
        ],
        out_shape=[
            jax.ShapeDtypeStruct((T, H), f32),
            jax.ShapeDtypeStruct((T, H), bf16),
            jax.ShapeDtypeStruct((T, 1), jnp.int32),
            jax.ShapeDtypeStruct((T, 1), f32),
        ],
        scratch_shapes=[pltpu.VMEM((1, E_), f32)],
    )(xf, attn2, out_proj_w.T.astype(bf16), out_proj_b.reshape(1, H),
      ln2_g.reshape(1, H), ln2_b.reshape(1, H), router_w.T.astype(bf16))

    disp = pl.pallas_call(
        _dispatch_kernel,
        grid=(E_,),
        in_specs=[
            pl.BlockSpec((T, 1), lambda e: (0, 0)),
            pl.BlockSpec((T, H), lambda e: (0, 0)),
        ],
        out_specs=pl.BlockSpec((C, H), lambda e: (e, 0)),
        out_shape=jax.ShapeDtypeStruct((E_ * C, H), bf16),
    )(slot, h2b)

    eout = pl.pallas_call(
        _ffn_kernel,
        grid=(E_,),
        in_specs=[
            pl.BlockSpec((C, H), lambda e: (e, 0)),
            pl.BlockSpec((1, H, FF), lambda e: (e, 0, 0)),
            pl.BlockSpec((1, 1, FF), lambda e: (e, 0, 0)),
            pl.BlockSpec((1, FF, H), lambda e: (e, 0, 0)),
            pl.BlockSpec((1, 1, H), lambda e: (e, 0, 0)),
        ],
        out_specs=pl.BlockSpec((C, H), lambda e: (e, 0)),
        out_shape=jax.ShapeDtypeStruct((E_ * C, H), bf16),
    )(disp, w1.transpose(0, 2, 1).astype(bf16), b1.reshape(E_, 1, FF),
      w2.transpose(0, 2, 1).astype(bf16), b2.reshape(E_, 1, H))

    out = pl.pallas_call(
        _combine_kernel,
        grid=(T // 512,),
        in_specs=[
            pl.BlockSpec((512, H), lambda i: (i, 0)),
            pl.BlockSpec((512, 1), lambda i: (i, 0)),
            pl.BlockSpec((512, 1), lambda i: (i, 0)),
            pl.BlockSpec((E_ * C, H), lambda i: (0, 0)),
        ],
        out_specs=pl.BlockSpec((512, H), lambda i: (i, 0)),
        out_shape=jax.ShapeDtypeStruct((T, H), f32),
    )(x2, slot, gate, eout)

    return out.reshape(B, S, H)


# speedup vs baseline: 3.6364x; 1.3417x over previous
import jax
import jax.numpy as jnp
from jax.experimental import pallas as pl
from jax.experimental.pallas import tpu as pltpu

NH_ = 16
E_ = 8
BQ = 512
BK = 512


def _ln(x, g, b):
    m = jnp.mean(x, axis=-1, keepdims=True)
    xm = x - m
    v = jnp.mean(xm * xm, axis=-1, keepdims=True)
    return xm / jnp.sqrt(v + 1e-5) * g + b


def _qkv_kernel(x_ref, g_ref, b_ref, wT_ref, pb_ref, o_ref):
    h = _ln(x_ref[...], g_ref[...], b_ref[...])
    acc = jax.lax.dot(h.astype(jnp.bfloat16), wT_ref[...],
                      preferred_element_type=jnp.float32)
    o_ref[...] = (acc + pb_ref[...]).astype(jnp.bfloat16)


def _attn_kernel(q_ref, k_ref, v_ref, o_ref, s_scr, a_scr):
    j = pl.program_id(2)
    nc = k_ref.shape[0] // BK
    for hh in range(2):
        q = q_ref[:, hh * 64:(hh + 1) * 64]
        for c in range(nc):
            sl = slice(c * BK, (c + 1) * BK)

            @pl.when(c <= j)
            def _(c=c, sl=sl):
                k = k_ref[sl, hh * 64:(hh + 1) * 64]
                s = jax.lax.dot_general(
                    q, k, (((1,), (1,)), ((), ())),
                    preferred_element_type=jnp.float32) * 0.125
                row = j * BQ + jax.lax.broadcasted_iota(jnp.int32, s.shape, 0)
                col = c * BK + jax.lax.broadcasted_iota(jnp.int32, s.shape, 1)
                s_scr[:, sl] = jnp.where(col <= row, s, -1e9)

            @pl.when(c > j)
            def _(sl=sl):
                s_scr[:, sl] = jnp.full((BQ, BK), -1e9, jnp.float32)

        s = s_scr[...]
        m = jnp.max(s, axis=-1, keepdims=True)
        p = jnp.exp(s - m)
        r = 1.0 / jnp.sum(p, axis=-1, keepdims=True)
        probs = (p * r).astype(jnp.bfloat16)
        for c in range(nc):
            sl = slice(c * BK, (c + 1) * BK)
            if c == 0:
                a_scr[...] = jax.lax.dot(
                    probs[:, sl], v_ref[sl, hh * 64:(hh + 1) * 64],
                    preferred_element_type=jnp.float32)
            else:
                @pl.when(c <= j)
                def _(sl=sl):
                    a_scr[...] += jax.lax.dot(
                        probs[:, sl], v_ref[sl, hh * 64:(hh + 1) * 64],
                        preferred_element_type=jnp.float32)

        o_ref[:, hh * 64:(hh + 1) * 64] = a_scr[...].astype(jnp.bfloat16)


def _proj_route_kernel(x_ref, a_ref, woT_ref, ob_ref, g2_ref, b2_ref, rT_ref,
                       x2_ref, h2b_ref, slot_ref, gate_ref, cnt_ref):
    i = pl.program_id(0)

    @pl.when(i == 0)
    def _():
        cnt_ref[...] = jnp.zeros_like(cnt_ref)

    x2 = x_ref[...] + (jax.lax.dot(a_ref[...], woT_ref[...],
                                   preferred_element_type=jnp.float32)
                       + ob_ref[...])
    x2_ref[...] = x2
    h2 = _ln(x2, g2_ref[...], b2_ref[...])
    h2b = h2.astype(jnp.bfloat16)
    h2b_ref[...] = h2b
    logits = jax.lax.dot(h2b, rT_ref[...],
                         preferred_element_type=jnp.float32)
    m = jnp.max(logits, axis=-1, keepdims=True)
    p = jnp.exp(logits - m)
    gp = p / jnp.sum(p, axis=-1, keepdims=True)
    gmax = jnp.max(gp, axis=-1, keepdims=True)
    e_io = jax.lax.broadcasted_iota(jnp.int32, gp.shape, 1)
    idx = jnp.min(jnp.where(gp == gmax, e_io, E_), axis=-1, keepdims=True)
    oh = (e_io == idx).astype(jnp.float32)
    r_io = jax.lax.broadcasted_iota(jnp.int32, (512, 512), 0)
    c_io = jax.lax.broadcasted_iota(jnp.int32, (512, 512), 1)
    tril = (c_io <= r_io).astype(jnp.bfloat16)
    cum = jax.lax.dot(tril, oh.astype(jnp.bfloat16),
                      preferred_element_type=jnp.float32)
    carry = cnt_ref[...]
    pos = jnp.sum((cum - 1.0 + carry) * oh, axis=-1, keepdims=True)
    pos = pos.astype(jnp.int32)
    cnt_ref[...] = carry + jnp.sum(oh, axis=0, keepdims=True)
    keep = pos < 512
    slot_ref[...] = jnp.where(keep, idx * 512 + pos, -1)
    gate_ref[...] = gmax


def _dispatch_kernel(slot_ref, h2b_ref, disp_ref):
    e = pl.program_id(0)
    slot = slot_ref[...]
    s_io = jax.lax.broadcasted_iota(jnp.int32, (slot.shape[0], 512), 1) + e * 512
    mask = (slot == s_io).astype(jnp.bfloat16)
    acc = jax.lax.dot_general(mask, h2b_ref[...],
                              (((0,), (0,)), ((), ())),
                              preferred_element_type=jnp.float32)
    disp_ref[...] = acc.astype(jnp.bfloat16)


def _ffn_kernel(disp_ref, w1_ref, b1_ref, w2_ref, b2_ref, eout_ref):
    mid = jax.lax.dot_general(disp_ref[...], w1_ref[0],
                              (((1,), (1,)), ((), ())),
                              preferred_element_type=jnp.float32) + b1_ref[0]
    mid = 0.5 * mid * (1.0 + jax.lax.erf(mid * (2.0 ** -0.5)))
    out = jax.lax.dot_general(mid.astype(jnp.bfloat16), w2_ref[0],
                              (((1,), (1,)), ((), ())),
                              preferred_element_type=jnp.float32) + b2_ref[0]
    eout_ref[...] = out.astype(jnp.bfloat16)


def _combine_kernel(x2_ref, slot_ref, gate_ref, eout_ref, o_ref):
    slot = slot_ref[...]
    s_io = jax.lax.broadcasted_iota(jnp.int32, (512, eout_ref.shape[0]), 1)
    mask = ((slot == s_io).astype(jnp.float32) * gate_ref[...]).astype(jnp.bfloat16)
    pick = jax.lax.dot(mask, eout_ref[...], preferred_element_type=jnp.float32)
    o_ref[...] = x2_ref[...] + pick


def kernel(x, ln1_g, ln1_b, in_proj_w, in_proj_b, out_proj_w, out_proj_b,
           ln2_g, ln2_b, router_w, w1, b1, w2, b2):
    B, S, H = x.shape
    T = B * S
    dh = H // NH_
    C = T // E_
    FF = w1.shape[1]
    f32 = jnp.float32
    bf16 = jnp.bfloat16
    xf = x.reshape(T, H)
    nj = S // BQ

    qkv = pl.pallas_call(
        _qkv_kernel,
        grid=(T // 512,),
        in_specs=[
            pl.BlockSpec((512, H), lambda i: (i, 0)),
            pl.BlockSpec((1, H), lambda i: (0, 0)),
            pl.BlockSpec((1, H), lambda i: (0, 0)),
            pl.BlockSpec((H, 3 * H), lambda i: (0, 0)),
            pl.BlockSpec((1, 3 * H), lambda i: (0, 0)),
        ],
        out_specs=pl.BlockSpec((512, 3 * H), lambda i: (i, 0)),
        out_shape=jax.ShapeDtypeStruct((T, 3 * H), bf16),
    )(xf, ln1_g.reshape(1, H), ln1_b.reshape(1, H),
      in_proj_w.T.astype(bf16), in_proj_b.reshape(1, 3 * H))

    attn = pl.pallas_call(
        _attn_kernel,
        grid=(B, NH_ // 2, nj),
        in_specs=[
            pl.BlockSpec((BQ, 128), lambda b, h2, j: (b * nj + j, h2)),
            pl.BlockSpec((S, 128), lambda b, h2, j: (b, NH_ // 2 + h2)),
            pl.BlockSpec((S, 128), lambda b, h2, j: (b, NH_ + h2)),
        ],
        out_specs=pl.BlockSpec((BQ, 128), lambda b, h2, j: (b * nj + j, h2)),
        out_shape=jax.ShapeDtypeStruct((T, H), bf16),
        scratch_shapes=[pltpu.VMEM((BQ, S), f32), pltpu.VMEM((BQ, dh), f32)],
    )(qkv, qkv, qkv)

    x2, h2b, slot, gate = pl.pallas_call(
        _proj_route_kernel,
        grid=(T // 512,),
        in_specs=[
            pl.BlockSpec((512, H), lambda i: (i, 0)),
            pl.BlockSpec((512, H), lambda i: (i, 0)),
            pl.BlockSpec((H, H), lambda i: (0, 0)),
            pl.BlockSpec((1, H), lambda i: (0, 0)),
            pl.BlockSpec((1, H), lambda i: (0, 0)),
            pl.BlockSpec((1, H), lambda i: (0, 0)),
            pl.BlockSpec((H, E_), lambda i: (0, 0)),
        ],
        out_specs=[
            pl.BlockSpec((512, H), lambda i: (i, 0)),
            pl.BlockSpec((512, H), lambda i: (i, 0)),
            pl.BlockSpec((512, 1), lambda i: (i, 0)),
            pl.BlockSpec((512, 1), lambda i: (i, 0)),
        ],
        out_shape=[
            jax.ShapeDtypeStruct((T, H), f32),
            jax.ShapeDtypeStruct((T, H), bf16),
            jax.ShapeDtypeStruct((T, 1), jnp.int32),
            jax.ShapeDtypeStruct((T, 1), f32),
        ],
        scratch_shapes=[pltpu.VMEM((1, E_), f32)],
    )(xf, attn, out_proj_w.T.astype(bf16), out_proj_b.reshape(1, H),
      ln2_g.reshape(1, H), ln2_b.reshape(1, H), router_w.T.astype(bf16))

    disp = pl.pallas_call(
        _dispatch_kernel,
        grid=(E_,),
        in_specs=[
            pl.BlockSpec((T, 1), lambda e: (0, 0)),
            pl.BlockSpec((T, H), lambda e: (0, 0)),
        ],
        out_specs=pl.BlockSpec((C, H), lambda e: (e, 0)),
        out_shape=jax.ShapeDtypeStruct((E_ * C, H), bf16),
    )(slot, h2b)

    eout = pl.pallas_call(
        _ffn_kernel,
        grid=(E_,),
        in_specs=[
            pl.BlockSpec((C, H), lambda e: (e, 0)),
            pl.BlockSpec((1, FF, H), lambda e: (e, 0, 0)),
            pl.BlockSpec((1, 1, FF), lambda e: (e, 0, 0)),
            pl.BlockSpec((1, H, FF), lambda e: (e, 0, 0)),
            pl.BlockSpec((1, 1, H), lambda e: (e, 0, 0)),
        ],
        out_specs=pl.BlockSpec((C, H), lambda e: (e, 0)),
        out_shape=jax.ShapeDtypeStruct((E_ * C, H), bf16),
    )(disp, w1.astype(bf16), b1.reshape(E_, 1, FF),
      w2.astype(bf16), b2.reshape(E_, 1, H))

    out = pl.pallas_call(
        _combine_kernel,
        grid=(T // 512,),
        in_specs=[
            pl.BlockSpec((512, H), lambda i: (i, 0)),
            pl.BlockSpec((512, 1), lambda i: (i, 0)),
            pl.BlockSpec((512, 1), lambda i: (i, 0)),
            pl.BlockSpec((E_ * C, H), lambda i: (0, 0)),
        ],
        out_specs=pl.BlockSpec((512, H), lambda i: (i, 0)),
        out_shape=jax.ShapeDtypeStruct((T, H), f32),
    )(x2, slot, gate, eout)

    return out.reshape(B, S, H)


# speedup vs baseline: 4.2385x; 1.1656x over previous
import functools

import jax
import jax.numpy as jnp
from jax.experimental import pallas as pl
from jax.experimental.pallas import tpu as pltpu
from jax.experimental.pallas import tpu_sc as plsc

NH_ = 16
E_ = 8
SC_W = 128
BQ = 512
BK = 512


def _ln(x, g, b):
    m = jnp.mean(x, axis=-1, keepdims=True)
    xm = x - m
    v = jnp.mean(xm * xm, axis=-1, keepdims=True)
    return xm / jnp.sqrt(v + 1e-5) * g + b


def _qkv_kernel(x_ref, g_ref, b_ref, wT_ref, pb_ref, o_ref):
    h = _ln(x_ref[...], g_ref[...], b_ref[...])
    acc = jax.lax.dot(h.astype(jnp.bfloat16), wT_ref[...],
                      preferred_element_type=jnp.float32)
    o_ref[...] = (acc + pb_ref[...]).astype(jnp.bfloat16)


def _attn_kernel(q_ref, k_ref, v_ref, o_ref,
                 qbd_scr, vbd_scr, s_scr, pp_scr, mx_scr, a_scr):
    j = pl.program_id(2)
    nc = k_ref.shape[0] // BK
    lane = jax.lax.broadcasted_iota(jnp.int32, (BQ, 128), 1)
    qf = q_ref[...].astype(jnp.float32)
    qbd_scr[0:BQ, :] = jnp.where(lane < 64, qf, 0.0).astype(jnp.bfloat16)
    qbd_scr[BQ:2 * BQ, :] = jnp.where(lane >= 64, qf, 0.0).astype(jnp.bfloat16)

    @pl.when(j == 0)
    def _():
        mx_scr[0:BQ, :] = jnp.full((BQ, 2 * nc), -1e9, jnp.float32)
        mx_scr[BQ:2 * BQ, :] = jnp.zeros((BQ, 2 * nc), jnp.float32)

    for c in range(nc):
        @pl.when(c <= j)
        def _(c=c):
            k = k_ref[c * BK:(c + 1) * BK, :]
            s2 = jax.lax.dot_general(
                qbd_scr[...], k, (((1,), (1,)), ((), ())),
                preferred_element_type=jnp.float32) * 0.125
            sa = s2[0:BQ, :]
            sb = s2[BQ:2 * BQ, :]
            row = j * BQ + jax.lax.broadcasted_iota(jnp.int32, (BQ, BK), 0)
            col = c * BK + jax.lax.broadcasted_iota(jnp.int32, (BQ, BK), 1)
            keep = col <= row

            @pl.when(c == j)
            def _():
                s_scr[:, 1024 * c:1024 * c + BK] = jnp.where(keep, sa, -1e9)
                s_scr[:, 1024 * c + BK:1024 * (c + 1)] = jnp.where(keep, sb, -1e9)

            @pl.when(c < j)
            def _():
                s_scr[:, 1024 * c:1024 * c + BK] = sa
                s_scr[:, 1024 * c + BK:1024 * (c + 1)] = sb
            ssa = s_scr[:, 1024 * c:1024 * c + BK]
            ssb = s_scr[:, 1024 * c + BK:1024 * (c + 1)]
            mx_scr[0:BQ, c:c + 1] = jnp.max(ssa, axis=-1, keepdims=True)
            mx_scr[0:BQ, nc + c:nc + c + 1] = jnp.max(ssb, axis=-1, keepdims=True)

    ma = jnp.max(mx_scr[0:BQ, 0:nc], axis=-1, keepdims=True)
    mb = jnp.max(mx_scr[0:BQ, nc:2 * nc], axis=-1, keepdims=True)
    for c in range(nc):
        @pl.when(c <= j)
        def _(c=c):
            pa = jnp.exp(s_scr[:, 1024 * c:1024 * c + BK] - ma)
            pb = jnp.exp(s_scr[:, 1024 * c + BK:1024 * (c + 1)] - mb)
            s_scr[:, 1024 * c:1024 * c + BK] = pa
            s_scr[:, 1024 * c + BK:1024 * (c + 1)] = pb
            mx_scr[BQ:2 * BQ, c:c + 1] = jnp.sum(pa, axis=-1, keepdims=True)
            mx_scr[BQ:2 * BQ, nc + c:nc + c + 1] = jnp.sum(pb, axis=-1, keepdims=True)

    ra = 1.0 / jnp.sum(mx_scr[BQ:2 * BQ, 0:nc], axis=-1, keepdims=True)
    rb = 1.0 / jnp.sum(mx_scr[BQ:2 * BQ, nc:2 * nc], axis=-1, keepdims=True)
    lane2 = jax.lax.broadcasted_iota(jnp.int32, (BK, 128), 1)
    for c in range(nc):
        @pl.when(c <= j)
        def _(c=c):
            pp_scr[:, 0:BK] = (s_scr[:, 1024 * c:1024 * c + BK] * ra).astype(jnp.bfloat16)
            pp_scr[:, BK:2 * BK] = (s_scr[:, 1024 * c + BK:1024 * (c + 1)] * rb).astype(jnp.bfloat16)
            vf = v_ref[c * BK:(c + 1) * BK, :].astype(jnp.float32)
            vbd_scr[0:BK, :] = jnp.where(lane2 < 64, vf, 0.0).astype(jnp.bfloat16)
            vbd_scr[BK:2 * BK, :] = jnp.where(lane2 >= 64, vf, 0.0).astype(jnp.bfloat16)
            part = jax.lax.dot(pp_scr[...], vbd_scr[...],
                               preferred_element_type=jnp.float32)
            if c == 0:
                a_scr[...] = part
            else:
                a_scr[...] += part

    o_ref[...] = a_scr[...].astype(jnp.bfloat16)


def _proj_route_kernel(x_ref, a_ref, woT_ref, ob_ref, g2_ref, b2_ref, rT_ref,
                       x2_ref, h2r_ref, slot_s_ref, slot_c_ref, gate_ref,
                       cnt_ref):
    i = pl.program_id(0)

    @pl.when(i == 0)
    def _():
        cnt_ref[...] = jnp.zeros_like(cnt_ref)

    x2 = x_ref[...] + (jax.lax.dot(a_ref[...], woT_ref[...],
                                   preferred_element_type=jnp.float32)
                       + ob_ref[...])
    x2_ref[...] = x2
    h2 = _ln(x2, g2_ref[...], b2_ref[...])
    h2b = h2.astype(jnp.bfloat16)
    h2r_ref[...] = h2b.astype(jnp.float32)
    logits = jax.lax.dot(h2b, rT_ref[...],
                         preferred_element_type=jnp.float32)
    m = jnp.max(logits, axis=-1, keepdims=True)
    p = jnp.exp(logits - m)
    gp = p / jnp.sum(p, axis=-1, keepdims=True)
    gmax = jnp.max(gp, axis=-1, keepdims=True)
    e_io = jax.lax.broadcasted_iota(jnp.int32, gp.shape, 1)
    idx = jnp.min(jnp.where(gp == gmax, e_io, E_), axis=-1, keepdims=True)
    oh = (e_io == idx).astype(jnp.float32)
    r_io = jax.lax.broadcasted_iota(jnp.int32, (512, 512), 0)
    c_io = jax.lax.broadcasted_iota(jnp.int32, (512, 512), 1)
    tril = (c_io <= r_io).astype(jnp.bfloat16)
    cum = jax.lax.dot(tril, oh.astype(jnp.bfloat16),
                      preferred_element_type=jnp.float32)
    carry = cnt_ref[...]
    pos = jnp.sum((cum - 1.0 + carry) * oh, axis=-1, keepdims=True)
    pos = pos.astype(jnp.int32)
    cnt_ref[...] = carry + jnp.sum(oh, axis=0, keepdims=True)
    keep = pos < 512
    slotv = idx * 512 + pos
    slot_s = jnp.where(keep, slotv, E_ * 512)
    q_io = jax.lax.broadcasted_iota(jnp.int32, (512, 4), 1)
    slot_s_ref[...] = slot_s * 4 + q_io
    slot_c_ref[...] = jnp.where(keep, slotv, -1)
    gate_ref[...] = jnp.where(keep, gmax, 0.0)


def _sc_scatter_rows(src, idx_row, out_rows):
    n, h = src.shape
    mesh = plsc.VectorSubcoreMesh(core_axis_name="c", subcore_axis_name="s")

    @functools.partial(
        pl.kernel,
        out_type=jax.ShapeDtypeStruct((out_rows, h), src.dtype),
        mesh=mesh)
    def k(x_hbm, i_hbm, o_hbm):
        def body(x_vmem, i_vmem):
            pltpu.sync_copy(x_vmem, o_hbm.at[i_vmem.at[0]])

        pltpu.emit_pipeline(
            body,
            grid=(n // SC_W,),
            in_specs=[pl.BlockSpec((SC_W, h), lambda i: (i, 0)),
                      pl.BlockSpec((1, SC_W), lambda i: (0, i))],
            out_specs=[],
            core_axis_name="s",
            dimension_semantics=(pltpu.PARALLEL,),
        )(x_hbm, i_hbm)

    return k(src, idx_row)


def _sc_scatter_rows(src, idx_row, out_rows):
    n, h = src.shape
    mesh = plsc.VectorSubcoreMesh(core_axis_name="c", subcore_axis_name="s")

    @functools.partial(
        pl.kernel,
        out_type=jax.ShapeDtypeStruct((out_rows, h), src.dtype),
        mesh=mesh)
    def k(x_hbm, i_hbm, o_hbm):
        def body(x_vmem, i_vmem):
            pltpu.sync_copy(x_vmem, o_hbm.at[i_vmem.at[0]])

        pltpu.emit_pipeline(
            body,
            grid=(n // SC_W,),
            in_specs=[pl.BlockSpec((SC_W, h), lambda i: (i, 0)),
                      pl.BlockSpec((1, SC_W), lambda i: (0, i))],
            out_specs=[],
            core_axis_name="s",
            dimension_semantics=(pltpu.PARALLEL,),
        )(x_hbm, i_hbm)

    return k(src, idx_row)


def _sc_gather_rows(table, idx_row):
    _, h = table.shape
    n = idx_row.shape[1]
    mesh = plsc.VectorSubcoreMesh(core_axis_name="c", subcore_axis_name="s")

    @functools.partial(
        pl.kernel,
        out_type=jax.ShapeDtypeStruct((n, h), table.dtype),
        mesh=mesh)
    def k(x_hbm, i_hbm, o_hbm):
        def body(i_vmem, o_vmem):
            pltpu.sync_copy(x_hbm.at[i_vmem.at[0]], o_vmem)

        pltpu.emit_pipeline(
            body,
            grid=(n // SC_W,),
            in_specs=[pl.BlockSpec((1, SC_W), lambda i: (0, i))],
            out_specs=[pl.BlockSpec((SC_W, h), lambda i: (i, 0))],
            core_axis_name="s",
            dimension_semantics=(pltpu.PARALLEL,),
        )(x_hbm, i_hbm)

    return k(table, idx_row)


def _ffn_kernel(disp_ref, w1_ref, b1_ref, w2_ref, b2_ref, eout_ref, acc_scr):
    f = pl.program_id(1)
    d = disp_ref[...]
    d = jnp.where(jnp.abs(d) < 1e30, d, 0.0)
    mid = jax.lax.dot_general(d.astype(jnp.bfloat16),
                              w1_ref[0].astype(jnp.bfloat16),
                              (((1,), (1,)), ((), ())),
                              preferred_element_type=jnp.float32) + b1_ref[0]
    mid = 0.5 * mid * (1.0 + jax.lax.erf(mid * (2.0 ** -0.5)))
    part = jax.lax.dot_general(mid.astype(jnp.bfloat16),
                               w2_ref[0].astype(jnp.bfloat16),
                               (((1,), (1,)), ((), ())),
                               preferred_element_type=jnp.float32)

    @pl.when(f == 0)
    def _():
        acc_scr[...] = part

    @pl.when(f == 1)
    def _():
        eout_ref[...] = (acc_scr[...] + part + b2_ref[0]).astype(jnp.bfloat16)


def _combine_kernel(x2_ref, slot_ref, gate_ref, eout_ref, o_ref):
    slot = slot_ref[...]
    s_io = jax.lax.broadcasted_iota(jnp.int32, (512, eout_ref.shape[0]), 1)
    mask = ((slot == s_io).astype(jnp.float32) * gate_ref[...]).astype(jnp.bfloat16)
    pick = jax.lax.dot(mask, eout_ref[...], preferred_element_type=jnp.float32)
    o_ref[...] = x2_ref[...] + pick


def kernel(x, ln1_g, ln1_b, in_proj_w, in_proj_b, out_proj_w, out_proj_b,
           ln2_g, ln2_b, router_w, w1, b1, w2, b2):
    B, S, H = x.shape
    T = B * S
    dh = H // NH_
    C = T // E_
    FF = w1.shape[1]
    f32 = jnp.float32
    bf16 = jnp.bfloat16
    xf = x.reshape(T, H)
    nj = S // BQ

    qkv = pl.pallas_call(
        _qkv_kernel,
        grid=(T // 512,),
        in_specs=[
            pl.BlockSpec((512, H), lambda i: (i, 0)),
            pl.BlockSpec((1, H), lambda i: (0, 0)),
            pl.BlockSpec((1, H), lambda i: (0, 0)),
            pl.BlockSpec((H, 3 * H), lambda i: (0, 0)),
            pl.BlockSpec((1, 3 * H), lambda i: (0, 0)),
        ],
        out_specs=pl.BlockSpec((512, 3 * H), lambda i: (i, 0)),
        out_shape=jax.ShapeDtypeStruct((T, 3 * H), bf16),
    )(xf, ln1_g.reshape(1, H), ln1_b.reshape(1, H),
      in_proj_w.T.astype(bf16), in_proj_b.reshape(1, 3 * H))

    attn = pl.pallas_call(
        _attn_kernel,
        grid=(B, NH_ // 2, nj),
        in_specs=[
            pl.BlockSpec((BQ, 128), lambda b, h2, j: (b * nj + j, h2)),
            pl.BlockSpec((S, 128), lambda b, h2, j: (b, NH_ // 2 + h2)),
            pl.BlockSpec((S, 128), lambda b, h2, j: (b, NH_ + h2)),
        ],
        out_specs=pl.BlockSpec((BQ, 128), lambda b, h2, j: (b * nj + j, h2)),
        out_shape=jax.ShapeDtypeStruct((T, H), bf16),
        scratch_shapes=[
            pltpu.VMEM((2 * BQ, 128), bf16),
            pltpu.VMEM((2 * BK, 128), bf16),
            pltpu.VMEM((BQ, 2 * S), f32),
            pltpu.VMEM((BQ, 2 * BK), bf16),
            pltpu.VMEM((2 * BQ, 8), f32),
            pltpu.VMEM((BQ, 128), f32),
        ],
    )(qkv, qkv, qkv)

    x2, h2b, slot_s, slot_c, gate = pl.pallas_call(
        _proj_route_kernel,
        grid=(T // 512,),
        in_specs=[
            pl.BlockSpec((512, H), lambda i: (i, 0)),
            pl.BlockSpec((512, H), lambda i: (i, 0)),
            pl.BlockSpec((H, H), lambda i: (0, 0)),
            pl.BlockSpec((1, H), lambda i: (0, 0)),
            pl.BlockSpec((1, H), lambda i: (0, 0)),
            pl.BlockSpec((1, H), lambda i: (0, 0)),
            pl.BlockSpec((H, E_), lambda i: (0, 0)),
        ],
        out_specs=[
            pl.BlockSpec((512, H), lambda i: (i, 0)),
            pl.BlockSpec((512, H), lambda i: (i, 0)),
            pl.BlockSpec((512, 4), lambda i: (i, 0)),
            pl.BlockSpec((512, 1), lambda i: (i, 0)),
            pl.BlockSpec((512, 1), lambda i: (i, 0)),
        ],
        out_shape=[
            jax.ShapeDtypeStruct((T, H), f32),
            jax.ShapeDtypeStruct((T, H), f32),
            jax.ShapeDtypeStruct((T, 4), jnp.int32),
            jax.ShapeDtypeStruct((T, 1), jnp.int32),
            jax.ShapeDtypeStruct((T, 1), f32),
        ],
        scratch_shapes=[pltpu.VMEM((1, E_), f32)],
    )(xf, attn, out_proj_w.T.astype(bf16), out_proj_b.reshape(1, H),
      ln2_g.reshape(1, H), ln2_b.reshape(1, H), router_w.T.astype(bf16))

    disp4 = _sc_scatter_rows(h2b.reshape(4 * T, H // 4),
                             slot_s.reshape(1, 4 * T), 4 * (E_ * C + 512))
    disp = disp4.reshape(E_ * C + 512, H)

    eout = pl.pallas_call(
        _ffn_kernel,
        grid=(E_, 2),
        in_specs=[
            pl.BlockSpec((C, H), lambda e, f: (e, 0)),
            pl.BlockSpec((1, FF // 2, H), lambda e, f: (e, f, 0)),
            pl.BlockSpec((1, 1, FF // 2), lambda e, f: (e, 0, f)),
            pl.BlockSpec((1, H, FF // 2), lambda e, f: (e, 0, f)),
            pl.BlockSpec((1, 1, H), lambda e, f: (e, 0, 0)),
        ],
        out_specs=pl.BlockSpec((C, H), lambda e, f: (e, 0)),
        out_shape=jax.ShapeDtypeStruct((E_ * C, H), bf16),
        scratch_shapes=[pltpu.VMEM((C, H), f32)],
    )(disp, w1, b1.reshape(E_, 1, FF), w2, b2.reshape(E_, 1, H))

    out = pl.pallas_call(
        _combine_kernel,
        grid=(T // 512,),
        in_specs=[
            pl.BlockSpec((512, H), lambda i: (i, 0)),
            pl.BlockSpec((512, 1), lambda i: (i, 0)),
            pl.BlockSpec((512, 1), lambda i: (i, 0)),
            pl.BlockSpec((E_ * C, H), lambda i: (0, 0)),
        ],
        out_specs=pl.BlockSpec((512, H), lambda i: (i, 0)),
        out_shape=jax.ShapeDtypeStruct((T, H), f32),
    )(x2, slot_c, gate, eout)

    return out.reshape(B, S, H)
